# baseline (speedup 1.0000x reference)
import functools

import jax
import jax.numpy as jnp
from jax import lax
from jax.experimental import pallas as pl
from jax.experimental.pallas import tpu as pltpu
from jax.experimental.pallas import tpu_sc as plsc

NC = 2
NS = 16
L = 16

B = 16384
BL = B // NC
D = 64
V = 100000

CW = 256
NCHUNK = (V + CW - 1) // CW
LAST_CW = V - (NCHUNK - 1) * CW
NK_EXTRA = NCHUNK - 16 * (NCHUNK // 16)
PARK = BL
IDXBLK = 2048
CHCAP = 4096

_mesh = plsc.VectorSubcoreMesh(core_axis_name="c", subcore_axis_name="s")


@functools.partial(
    pl.kernel,
    mesh=_mesh,
    out_type=jax.ShapeDtypeStruct((B,), jnp.float32),
    compiler_params=pltpu.CompilerParams(
        needs_layout_passes=False, use_tc_tiling_on_sc=True),
    scratch_types=[
        pltpu.VMEM((IDXBLK,), jnp.int32),
        pltpu.VMEM((64, 128), jnp.int32),
        pltpu.VMEM((64, 64), jnp.int32),
        pltpu.VMEM((32, 128), jnp.int32),
        pltpu.VMEM((D, CW), jnp.float32),
        pltpu.VMEM((1, CW), jnp.float32),
        pltpu.VMEM((64, 128), jnp.float32),
        pltpu.VMEM((64, 128), jnp.float32),
        pltpu.VMEM((8, 128), jnp.float32),
        pltpu.VMEM((64,), jnp.float32),
        pltpu.VMEM_SHARED((BL + 64, 128), jnp.float32),
        pltpu.VMEM_SHARED((BL + 64,), jnp.float32),
    ],
)
def _mf_fwd(user_hbm, item_hbm, uet_hbm, iet_hbm, ubt_hbm, ibt_hbm,
            uet_tail, iet_tail, ubt_tail, ibt_tail, out_hbm,
            idxb, hp, chb, chu, chunk_v, bias_v, rows_v, irows_v,
            bounce, preds_v, sh_rows, sh_preds):
    c = lax.axis_index("c")
    s = lax.axis_index("s")
    lane = lax.iota(jnp.int32, L)
    onehot0 = (lane == 0).astype(jnp.float32)
    lane15 = lane == (L - 1)
    nk = jnp.where(s < NK_EXTRA, NCHUNK // 16 + 1, NCHUNK // 16)
    b0 = pl.multiple_of(c * BL, BL)

    def slab_filter(src_hbm):
        def blk(t, pos_t):
            pltpu.sync_copy(src_hbm.at[pl.ds(b0 + t * IDXBLK, IDXBLK)], idxb)
            def body(g, pos0):
                for kk in range(4):
                    off = pl.multiple_of(g * 64 + kk * 16, 16)
                    u = idxb[pl.ds(off, 16)]
                    m = ((u >> 8) & 15) == s
                    mi = jnp.where(m, 1, 0)
                    cs = plsc.cumsum(mi)
                    pos = pos0 + cs - mi
                    p = ((t * IDXBLK + off + lane) << 17) | u
                    plsc.store_scatter(hp, [pos >> 7, pos & 127], p, mask=m)
                    pos0 = pos0 + jnp.max(cs)
                return pos0
            return lax.fori_loop(0, IDXBLK // 64, body, pos_t)
        return lax.fori_loop(0, BL // IDXBLK, blk, jnp.int32(0))

    def park(n):
        r0 = n >> 6
        rem = n & 63
        @pl.when(rem != 0)
        def _():
            for h in range(4):
                pj = 16 * h + lane
                plsc.store_scatter(chb, [jnp.full((L,), r0, jnp.int32), pj],
                                   jnp.full((L,), PARK, jnp.int32) + pj,
                                   mask=pj >= rem)

    def chunk_filter(k, nh):
        def body(g, pos0):
            co = pl.multiple_of((g & 7) * 16, 16)
            p = hp[g >> 3, pl.ds(co, 16)]
            u = p & 0x1FFFF
            m = ((u >> 12) == k) & ((g * 16 + lane) < nh)
            mi = jnp.where(m, 1, 0)
            cs = plsc.cumsum(mi)
            pos = jnp.minimum(pos0 + cs - mi, CHCAP - 1)
            plsc.store_scatter(chb, [pos >> 6, pos & 63], p >> 17, mask=m)
            plsc.store_scatter(chu, [pos >> 7, pos & 127], u & (CW - 1),
                               mask=m)
            return pos0 + jnp.max(cs)
        n = lax.fori_loop(0, (nh + 15) >> 4, body, jnp.int32(0))
        n = jnp.minimum(n, CHCAP)
        park(n)
        return n

    def load_chunk(tab_hbm, bt_hbm, tab_tail, bt_tail, cid):
        u0 = pl.multiple_of(cid * CW, CW)
        @pl.when(cid != NCHUNK - 1)
        def _():
            pltpu.sync_copy(tab_hbm.at[:, pl.ds(u0, CW)], chunk_v)
            pltpu.sync_copy(bt_hbm.at[:, pl.ds(u0, CW)], bias_v)
        @pl.when(cid == NCHUNK - 1)
        def _():
            pltpu.sync_copy(tab_tail, chunk_v)
            pltpu.sync_copy(bt_tail, bias_v)

    nh_i = slab_filter(item_hbm)

    def item_chunk(k, carry):
        cid = 16 * k + s
        nch = chunk_filter(k, nh_i)
        @pl.when(nch > 0)
        def _():
            load_chunk(iet_hbm, ibt_hbm, iet_tail, ibt_tail, cid)
            def batch(bb, carry2):
                def group(g, carry3):
                    fl = bb * 64 + g * 16
                    co = pl.multiple_of(fl & 127, 16)
                    ul = chu[fl >> 7, pl.ds(co, 16)]
                    m = (fl + lane) < nch
                    for d in range(D):
                        bounce[d >> 3, pl.ds((d & 7) * 16, 16)] = (
                            plsc.load_gather(
                                chunk_v,
                                [jnp.full((L,), d, jnp.int32), ul], mask=m))
                    for jj in range(L):
                        row = g * 16 + jj
                        for q in range(4):
                            fd = (q * 16 + lane) * 16 + jj
                            rv = plsc.load_gather(bounce, [fd >> 7, fd & 127])
                            rows_v[row, pl.ds(q * 16, 16)] = rv
                    bv = plsc.load_gather(bias_v,
                                          [jnp.zeros((L,), jnp.int32), ul],
                                          mask=m)
                    plsc.store_scatter(rows_v,
                                       [g * 16 + lane,
                                        jnp.full((L,), D, jnp.int32)], bv)
                    return carry3
                lax.fori_loop(0, 4, group, 0)
                pltpu.sync_copy(rows_v, sh_rows.at[chb.at[bb]])
                return carry2
            lax.fori_loop(0, (nch + 63) >> 6, batch, 0)
        return carry
    lax.fori_loop(0, nk, item_chunk, 0)

    plsc.subcore_barrier()

    nh_u = slab_filter(user_hbm)

    def user_chunk(k, carry):
        cid = 16 * k + s
        nch = chunk_filter(k, nh_u)
        @pl.when(nch > 0)
        def _():
            load_chunk(uet_hbm, ubt_hbm, uet_tail, ubt_tail, cid)
            def batch(bb, carry2):
                pltpu.sync_copy(sh_rows.at[chb.at[bb]], irows_v)
                def group(g, carry3):
                    fl = bb * 64 + g * 16
                    co = pl.multiple_of(fl & 127, 16)
                    ul = chu[fl >> 7, pl.ds(co, 16)]
                    m = (fl + lane) < nch
                    for d in range(D):
                        bounce[d >> 3, pl.ds((d & 7) * 16, 16)] = (
                            plsc.load_gather(
                                chunk_v,
                                [jnp.full((L,), d, jnp.int32), ul], mask=m))
                    ub = plsc.load_gather(bias_v,
                                          [jnp.zeros((L,), jnp.int32), ul],
                                          mask=m)
                    for jj in range(L):
                        row = g * 16 + jj
                        acc = None
                        for q in range(4):
                            fd = (q * 16 + lane) * 16 + jj
                            uv = plsc.load_gather(bounce, [fd >> 7, fd & 127])
                            iv = irows_v[row, pl.ds(q * 16, 16)]
                            t = uv * iv
                            acc = t if acc is None else acc + t
                        ibv = irows_v[row, pl.ds(D, 16)] * onehot0
                        cs = plsc.cumsum(acc + ibv)
                        plsc.store_scatter(preds_v,
                                           [jnp.full((L,), row, jnp.int32)],
                                           cs, mask=lane15)
                    po = pl.multiple_of(g * 16, 16)
                    preds_v[pl.ds(po, 16)] = preds_v[pl.ds(po, 16)] + ub
                    return carry3
                lax.fori_loop(0, 4, group, 0)
                pltpu.sync_copy(preds_v, sh_preds.at[chb.at[bb]])
                return carry2
            lax.fori_loop(0, (nch + 63) >> 6, batch, 0)
        return carry
    lax.fori_loop(0, nk, user_chunk, 0)

    plsc.subcore_barrier()

    o0 = pl.multiple_of(s * (BL // NS), BL // NS)
    pltpu.sync_copy(sh_preds.at[pl.ds(o0, BL // NS)],
                    out_hbm.at[pl.ds(b0 + o0, BL // NS)])


def kernel(user, item, user_e, item_e, user_b, item_b):
    t0 = (NCHUNK - 1) * CW
    padw = ((0, 0), (0, CW - LAST_CW))
    return _mf_fwd(user.astype(jnp.int32), item.astype(jnp.int32),
                   user_e.T, item_e.T, user_b.T, item_b.T,
                   jnp.pad(user_e.T[:, t0:], padw),
                   jnp.pad(item_e.T[:, t0:], padw),
                   jnp.pad(user_b.T[:, t0:], padw),
                   jnp.pad(item_b.T[:, t0:], padw))


# speedup vs baseline: 1.3071x; 1.3071x over previous
import functools

import jax
import jax.numpy as jnp
from jax import lax
from jax.experimental import pallas as pl
from jax.experimental.pallas import tpu as pltpu
from jax.experimental.pallas import tpu_sc as plsc

NC = 2
NS = 16
L = 16

B = 16384
BL = B // NC
D = 64
V = 100000

CW = 256
NCHUNK = (V + CW - 1) // CW
LAST_CW = V - (NCHUNK - 1) * CW
NK_EXTRA = NCHUNK - 16 * (NCHUNK // 16)
NK_MAX = NCHUNK // 16 + 1
PARK = BL
IDXBLK = 1024
HCAP = 2048

_mesh = plsc.VectorSubcoreMesh(core_axis_name="c", subcore_axis_name="s")


@functools.partial(
    pl.kernel,
    mesh=_mesh,
    out_type=jax.ShapeDtypeStruct((B,), jnp.float32),
    compiler_params=pltpu.CompilerParams(
        needs_layout_passes=False, use_tc_tiling_on_sc=True),
    scratch_types=[
        pltpu.VMEM((IDXBLK,), jnp.int32),
        pltpu.VMEM((16, 128), jnp.int32),
        pltpu.VMEM((32, 64), jnp.int32),
        pltpu.VMEM((16, 128), jnp.int32),
        pltpu.VMEM((D, CW), jnp.float32),
        pltpu.VMEM((D, CW), jnp.float32),
        pltpu.VMEM((1, CW), jnp.float32),
        pltpu.VMEM((1, CW), jnp.float32),
        pltpu.VMEM((64, 128), jnp.float32),
        pltpu.VMEM((64, 128), jnp.float32),
        pltpu.VMEM((8, 128), jnp.float32),
        pltpu.VMEM((64,), jnp.float32),
        pltpu.SemaphoreType.DMA,
        pltpu.SemaphoreType.DMA,
        pltpu.VMEM_SHARED((BL + 64, 128), jnp.float32),
        pltpu.VMEM_SHARED((BL + 64,), jnp.float32),
    ],
)
def _mf_fwd(user_hbm, item_hbm, uet_hbm, iet_hbm, ubt_hbm, ibt_hbm,
            uet_tail, iet_tail, ubt_tail, ibt_tail, out_hbm,
            idxb, hp, chb, chu, chunk_a, chunk_b, bias_a, bias_b,
            rows_v, irows_v, bounce, preds_v, sem_a, sem_b,
            sh_rows, sh_preds):
    c = lax.axis_index("c")
    s = lax.axis_index("s")
    lane = lax.iota(jnp.int32, L)
    onehot0 = (lane == 0).astype(jnp.float32)
    lane15 = lane == (L - 1)
    nk = jnp.where(s < NK_EXTRA, NK_MAX, NK_MAX - 1)
    b0 = pl.multiple_of(c * BL, BL)

    def slab_filter(src_hbm):
        def blk(t, pos_t):
            pltpu.sync_copy(src_hbm.at[pl.ds(b0 + t * IDXBLK, IDXBLK)], idxb)
            def body(g, posv):
                for kk in range(4):
                    off = pl.multiple_of(g * 64 + kk * 16, 16)
                    u = idxb[pl.ds(off, 16)]
                    m = ((u >> 8) & 15) == s
                    mi = jnp.where(m, 1, 0)
                    cs = plsc.cumsum(mi)
                    pos = jnp.minimum(posv + cs - mi, HCAP - 1)
                    p = ((t * IDXBLK + off + lane) << 17) | u
                    plsc.store_scatter(hp, [pos >> 7, pos & 127], p, mask=m)
                    posv = posv + plsc.all_reduce_population_count(m)
                return posv
            return lax.fori_loop(0, IDXBLK // 64, body, pos_t)
        posv = lax.fori_loop(0, BL // IDXBLK, blk,
                             jnp.zeros((L,), jnp.int32))
        return jnp.minimum(jnp.max(posv), HCAP)

    def park(n):
        r0 = n >> 6
        rem = n & 63
        @pl.when(rem != 0)
        def _():
            for h in range(4):
                pj = 16 * h + lane
                plsc.store_scatter(chb, [jnp.full((L,), r0, jnp.int32), pj],
                                   jnp.full((L,), PARK, jnp.int32) + pj,
                                   mask=pj >= rem)

    def chunk_filter(k, nh):
        def body(g, posv):
            co = pl.multiple_of((g & 7) * 16, 16)
            p = hp[g >> 3, pl.ds(co, 16)]
            u = p & 0x1FFFF
            m = ((u >> 12) == k) & ((g * 16 + lane) < nh)
            mi = jnp.where(m, 1, 0)
            cs = plsc.cumsum(mi)
            pos = jnp.minimum(posv + cs - mi, HCAP - 1)
            plsc.store_scatter(chb, [pos >> 6, pos & 63], p >> 17, mask=m)
            plsc.store_scatter(chu, [pos >> 7, pos & 127], u & (CW - 1),
                               mask=m)
            return posv + plsc.all_reduce_population_count(m)
        posv = lax.fori_loop(0, (nh + 15) >> 4, body,
                             jnp.zeros((L,), jnp.int32))
        n = jnp.minimum(jnp.max(posv), HCAP)
        park(n)
        return n

    def start_chunk(tab_hbm, bt_hbm, tab_tail, bt_tail, k, bufc, bufb, sem):
        cid = 16 * k + s
        u0 = pl.multiple_of(cid * CW, CW)
        @pl.when(cid != NCHUNK - 1)
        def _():
            pltpu.async_copy(tab_hbm.at[:, pl.ds(u0, CW)], bufc, sem)
            pltpu.async_copy(bt_hbm.at[:, pl.ds(u0, CW)], bufb, sem)
        @pl.when(cid == NCHUNK - 1)
        def _():
            pltpu.async_copy(tab_tail, bufc, sem)
            pltpu.async_copy(bt_tail, bufb, sem)

    def wait_chunk(tab_hbm, bt_hbm, bufc, bufb, sem):
        pltpu.make_async_copy(tab_hbm.at[:, pl.ds(0, CW)], bufc, sem).wait()
        pltpu.make_async_copy(bt_hbm.at[:, pl.ds(0, CW)], bufb, sem).wait()

    def fill_bounce(bufc, bufb, bb, g, nch):
        fl = bb * 64 + g * 16
        co = pl.multiple_of(fl & 127, 16)
        ul = chu[fl >> 7, pl.ds(co, 16)]
        m = (fl + lane) < nch
        for d in range(D):
            bounce[d >> 3, pl.ds((d & 7) * 16, 16)] = plsc.load_gather(
                bufc, [jnp.full((L,), d, jnp.int32), ul], mask=m)
        bv = plsc.load_gather(bufb, [jnp.zeros((L,), jnp.int32), ul], mask=m)
        return bv

    def item_batches(bufc, bufb, nch):
        def batch(bb, carry2):
            def group(g, carry3):
                bv = fill_bounce(bufc, bufb, bb, g, nch)
                for jj in range(L):
                    row = g * 16 + jj
                    for q in range(4):
                        fd = (q * 16 + lane) * 16 + jj
                        rv = plsc.load_gather(bounce, [fd >> 7, fd & 127])
                        rows_v[row, pl.ds(q * 16, 16)] = rv
                plsc.store_scatter(rows_v, [g * 16 + lane,
                                            jnp.full((L,), D, jnp.int32)], bv)
                return carry3
            lax.fori_loop(0, 4, group, 0)
            pltpu.sync_copy(rows_v, sh_rows.at[chb.at[bb]])
            return carry2
        lax.fori_loop(0, (nch + 63) >> 6, batch, 0)

    def user_batches(bufc, bufb, nch):
        def batch(bb, carry2):
            pltpu.sync_copy(sh_rows.at[chb.at[bb]], irows_v)
            def group(g, carry3):
                ub = fill_bounce(bufc, bufb, bb, g, nch)
                for jj in range(L):
                    row = g * 16 + jj
                    acc = None
                    for q in range(4):
                        fd = (q * 16 + lane) * 16 + jj
                        uv = plsc.load_gather(bounce, [fd >> 7, fd & 127])
                        iv = irows_v[row, pl.ds(q * 16, 16)]
                        t = uv * iv
                        acc = t if acc is None else acc + t
                    ibv = irows_v[row, pl.ds(D, 16)] * onehot0
                    cs = plsc.cumsum(acc + ibv)
                    plsc.store_scatter(preds_v,
                                       [jnp.full((L,), row, jnp.int32)],
                                       cs, mask=lane15)
                po = pl.multiple_of(g * 16, 16)
                preds_v[pl.ds(po, 16)] = preds_v[pl.ds(po, 16)] + ub
                return carry3
            lax.fori_loop(0, 4, group, 0)
            pltpu.sync_copy(preds_v, sh_preds.at[chb.at[bb]])
            return carry2
        lax.fori_loop(0, (nch + 63) >> 6, batch, 0)

    def process_side(tab_hbm, bt_hbm, tab_tail, bt_tail, nh, batches):
        bufs = ((chunk_a, bias_a, sem_a), (chunk_b, bias_b, sem_b))
        start_chunk(tab_hbm, bt_hbm, tab_tail, bt_tail, 0,
                    chunk_a, bias_a, sem_a)
        def pair(kk, carry):
            for par in range(2):
                k = 2 * kk + par
                bufc, bufb, sem = bufs[par]
                obufc, obufb, osem = bufs[1 - par]
                @pl.when(k < nk)
                def _():
                    @pl.when(k + 1 < nk)
                    def _():
                        start_chunk(tab_hbm, bt_hbm, tab_tail, bt_tail,
                                    k + 1, obufc, obufb, osem)
                    nch = chunk_filter(k, nh)
                    wait_chunk(tab_hbm, bt_hbm, bufc, bufb, sem)
                    @pl.when(nch > 0)
                    def _():
                        batches(bufc, bufb, nch)
            return carry
        lax.fori_loop(0, (NK_MAX + 1) // 2, pair, 0)

    nh_i = slab_filter(item_hbm)
    process_side(iet_hbm, ibt_hbm, iet_tail, ibt_tail, nh_i, item_batches)

    plsc.subcore_barrier()

    nh_u = slab_filter(user_hbm)
    process_side(uet_hbm, ubt_hbm, uet_tail, ubt_tail, nh_u, user_batches)

    plsc.subcore_barrier()

    o0 = pl.multiple_of(s * (BL // NS), BL // NS)
    pltpu.sync_copy(sh_preds.at[pl.ds(o0, BL // NS)],
                    out_hbm.at[pl.ds(b0 + o0, BL // NS)])


def kernel(user, item, user_e, item_e, user_b, item_b):
    t0 = (NCHUNK - 1) * CW
    padw = ((0, 0), (0, CW - LAST_CW))
    return _mf_fwd(user.astype(jnp.int32), item.astype(jnp.int32),
                   user_e.T, item_e.T, user_b.T, item_b.T,
                   jnp.pad(user_e.T[:, t0:], padw),
                   jnp.pad(item_e.T[:, t0:], padw),
                   jnp.pad(user_b.T[:, t0:], padw),
                   jnp.pad(item_b.T[:, t0:], padw))


# speedup vs baseline: 2.3117x; 1.7686x over previous
import functools

import jax
import jax.numpy as jnp
from jax import lax
from jax.experimental import pallas as pl
from jax.experimental.pallas import tpu as pltpu
from jax.experimental.pallas import tpu_sc as plsc

NC = 2
NS = 16
L = 16
NW = NC * NS

B = 16384
D = 64
BPW = B // NW
CH = 128
NCH = BPW // CH

_mesh = plsc.VectorSubcoreMesh(core_axis_name="c", subcore_axis_name="s")


@functools.partial(
    pl.kernel,
    mesh=_mesh,
    out_type=jax.ShapeDtypeStruct((B,), jnp.float32),
    compiler_params=pltpu.CompilerParams(
        needs_layout_passes=False, use_tc_tiling_on_sc=False),
    scratch_types=[
        pltpu.VMEM((NCH, CH), jnp.int32),
        pltpu.VMEM((NCH, CH), jnp.int32),
        pltpu.VMEM((BPW, D), jnp.float32),
        pltpu.VMEM((BPW, D), jnp.float32),
        pltpu.VMEM((BPW,), jnp.float32),
        pltpu.VMEM((BPW,), jnp.float32),
        pltpu.VMEM((BPW,), jnp.float32),
        pltpu.SemaphoreType.DMA,
    ],
)
def _mf_fwd(user_hbm, item_hbm, ue_hbm, ie_hbm, ub_hbm, ib_hbm, out_hbm,
            uidx, iidx, urows, irows, ubias, ibias, outv, sem):
    wid = lax.axis_index("c") * NS + lax.axis_index("s")
    base = pl.multiple_of(wid * BPW, BPW)

    crow = pl.multiple_of(wid * NCH, NCH)
    pltpu.sync_copy(user_hbm.at[pl.ds(crow, NCH)], uidx)
    pltpu.sync_copy(item_hbm.at[pl.ds(crow, NCH)], iidx)

    copies = []
    for j in range(NCH):
        dst = pl.ds(j * CH, CH)
        copies.append(pltpu.async_copy(ue_hbm.at[uidx.at[j]], urows.at[dst], sem))
        copies.append(pltpu.async_copy(ie_hbm.at[iidx.at[j]], irows.at[dst], sem))
        copies.append(pltpu.async_copy(ub_hbm.at[uidx.at[j]], ubias.at[dst], sem))
        copies.append(pltpu.async_copy(ib_hbm.at[iidx.at[j]], ibias.at[dst], sem))
    for c in copies:
        c.wait()

    lane = lax.iota(jnp.int32, L)
    last_lane = lane == (L - 1)
    G = 16

    def body(g, carry):
        gb = pl.multiple_of(g * G, G)
        for k in range(G):
            b = gb + k
            acc = urows[b, pl.ds(0, L)] * irows[b, pl.ds(0, L)]
            for c in range(1, D // L):
                acc = acc + urows[b, pl.ds(c * L, L)] * irows[b, pl.ds(c * L, L)]
            total = plsc.cumsum(acc)
            idxv = jnp.full((L,), b, jnp.int32)
            plsc.store_scatter(outv, [idxv], total, mask=last_lane)
        return carry

    lax.fori_loop(0, BPW // G, body, 0)

    for g in range(BPW // L):
        s = pl.ds(g * L, L)
        outv[s] = outv[s] + ubias[s] + ibias[s]

    pltpu.sync_copy(outv, out_hbm.at[pl.ds(base, BPW)])


def kernel(user, item, user_e, item_e, user_b, item_b):
    u2 = user.astype(jnp.int32).reshape(B // CH, CH)
    i2 = item.astype(jnp.int32).reshape(B // CH, CH)
    return _mf_fwd(u2, i2, user_e, item_e,
                   user_b.reshape(-1), item_b.reshape(-1))
